# baseline (speedup 1.0000x reference)
import jax
import jax.numpy as jnp
import numpy as np
from jax import lax
from jax.experimental import pallas as pl
from jax.experimental.pallas import tpu as pltpu
from jax.experimental.pallas import tpu_sc as plsc

VOCAB = 100000
D_MODEL = 128
SEQ_LEN = 2048
BATCH = 4

_NC = 2
_NS = 16
_NW = _NC * _NS
_B_FLAT = BATCH * SEQ_LEN
_CHUNK = _B_FLAT // _NW
_NSUB = 4
_SUB = _CHUNK // _NSUB


def _sinusoidal_pe() -> np.ndarray:
    pos = np.arange(SEQ_LEN)[:, None].astype(np.float64)
    i = np.arange(D_MODEL)[None, :].astype(np.float64)
    angle = pos / np.power(10000.0, (2.0 * (i // 2)) / D_MODEL)
    pe = np.zeros((SEQ_LEN, D_MODEL), dtype=np.float32)
    pe[:, 0::2] = np.sin(angle[:, 0::2])
    pe[:, 1::2] = np.cos(angle[:, 1::2])
    return pe


_PE = _sinusoidal_pe()


def _embed_body(w_hbm, x1_hbm, x2_hbm, pe_hbm, out1_hbm, out2_hbm,
                idx1_v, idx2_v, rows1_v, rows2_v, *sems):
    wid = lax.axis_index("s") * _NC + lax.axis_index("c")
    base = wid * _CHUNK
    b = base // SEQ_LEN
    s0 = lax.rem(base, SEQ_LEN)

    pltpu.sync_copy(x1_hbm.at[b, pl.ds(s0, _CHUNK)], idx1_v)
    pltpu.sync_copy(x2_hbm.at[b, pl.ds(s0, _CHUNK)], idx2_v)

    pe_cps = []
    for j in range(_NSUB):
        src = pe_hbm.at[pl.ds(s0 + _SUB * j, _SUB)]
        for k, rows_v in enumerate((rows1_v, rows2_v)):
            sem = sems[2 * j + k]
            pe_cps.append((pltpu.async_copy(src, rows_v.at[j], sem), sem))

    gathers = []
    for j in range(_NSUB):
        isl = pl.ds(_SUB * j, _SUB)
        for k, (idx_v, rows_v) in enumerate(((idx1_v, rows1_v),
                                             (idx2_v, rows2_v))):
            cp, sem = pe_cps[2 * j + k]
            cp.wait()
            gathers.append(
                (pltpu.async_copy(w_hbm.at[idx_v.at[isl]], rows_v.at[j],
                                  sem, add=True), sem))

    stores = []
    for j in range(_NSUB):
        for k, (rows_v, out_hbm) in enumerate(((rows1_v, out1_hbm),
                                               (rows2_v, out2_hbm))):
            cp, sem = gathers[2 * j + k]
            cp.wait()
            o0 = s0 + _SUB * j
            stores.append(pltpu.async_copy(
                rows_v.at[j], out_hbm.at[b, pl.ds(o0, _SUB)], sem))
    for st in stores:
        st.wait()


_sc_embed = pl.kernel(
    _embed_body,
    out_type=(
        jax.ShapeDtypeStruct((BATCH, SEQ_LEN, D_MODEL), jnp.float32),
        jax.ShapeDtypeStruct((BATCH, SEQ_LEN, D_MODEL), jnp.float32),
    ),
    mesh=plsc.VectorSubcoreMesh(core_axis_name="c", subcore_axis_name="s"),
    scratch_types=[
        pltpu.VMEM((_CHUNK,), jnp.int32),
        pltpu.VMEM((_CHUNK,), jnp.int32),
        pltpu.VMEM((_NSUB, _SUB, D_MODEL), jnp.float32),
        pltpu.VMEM((_NSUB, _SUB, D_MODEL), jnp.float32),
    ] + [pltpu.SemaphoreType.DMA] * (2 * _NSUB),
)


@jax.jit
def kernel(x, x_output, W):
    pe = jnp.asarray(_PE)
    return _sc_embed(W, x, x_output, pe)


# speedup vs baseline: 1.0597x; 1.0597x over previous
import jax
import jax.numpy as jnp
import numpy as np
from jax import lax
from jax.experimental import pallas as pl
from jax.experimental.pallas import tpu as pltpu
from jax.experimental.pallas import tpu_sc as plsc

VOCAB = 100000
D_MODEL = 128
SEQ_LEN = 2048
BATCH = 4

_NC = 2
_NS = 16
_NW = _NC * _NS
_B_FLAT = BATCH * SEQ_LEN
_CHUNK = _B_FLAT // _NW
_NSUB = 4
_SUB = _CHUNK // _NSUB
_DV = D_MODEL // 16


def _build_pe() -> np.ndarray:
    pos = np.arange(SEQ_LEN)[:, None].astype(np.float64)
    i = np.arange(D_MODEL)[None, :].astype(np.float64)
    angle = pos / np.power(10000.0, (2.0 * (i // 2)) / D_MODEL)
    pe = np.zeros((SEQ_LEN, D_MODEL), dtype=np.float32)
    pe[:, 0::2] = np.sin(angle[:, 0::2])
    pe[:, 1::2] = np.cos(angle[:, 1::2])
    return pe


_PE = _build_pe()


def _embed_body(w_hbm, x1_hbm, x2_hbm, pe_hbm, out1_hbm, out2_hbm,
                idx1_v, idx2_v, pe_v, rows1_v, rows2_v, *sems):
    wid = lax.axis_index("s") * _NC + lax.axis_index("c")
    base = wid * _CHUNK
    b = base // SEQ_LEN
    s0 = lax.rem(base, SEQ_LEN)

    cp_pe = pltpu.async_copy(pe_hbm.at[pl.ds(s0, _CHUNK)], pe_v, sems[0])
    cp_i1 = pltpu.async_copy(x1_hbm.at[b, pl.ds(s0, _CHUNK)], idx1_v, sems[1])
    cp_i2 = pltpu.async_copy(x2_hbm.at[b, pl.ds(s0, _CHUNK)], idx2_v, sems[2])
    cp_i1.wait()
    cp_i2.wait()

    gathers = []
    for j in range(_NSUB):
        sl = pl.ds(_SUB * j, _SUB)
        for idx_v, rows_v in ((idx1_v, rows1_v), (idx2_v, rows2_v)):
            sem = sems[3 + len(gathers)]
            gathers.append(
                (pltpu.async_copy(w_hbm.at[idx_v.at[sl]], rows_v.at[sl], sem),
                 sem))

    cp_pe.wait()
    stores = []
    for j in range(_NSUB):
        sl = pl.ds(_SUB * j, _SUB)
        (cp1, sem1), (cp2, sem2) = gathers[2 * j], gathers[2 * j + 1]
        cp1.wait()
        cp2.wait()

        @plsc.parallel_loop(_SUB * j, _SUB * (j + 1), step=4)
        def _(r):
            for rr in range(4):
                for d in range(_DV):
                    dsl = pl.ds(d * 16, 16)
                    pv = pe_v[r + rr, dsl]
                    plsc.addupdate(rows1_v.at[r + rr, dsl], pv)
                    plsc.addupdate(rows2_v.at[r + rr, dsl], pv)

        o0 = s0 + _SUB * j
        stores.append(pltpu.async_copy(
            rows1_v.at[sl], out1_hbm.at[b, pl.ds(o0, _SUB)], sem1))
        stores.append(pltpu.async_copy(
            rows2_v.at[sl], out2_hbm.at[b, pl.ds(o0, _SUB)], sem2))
    for st in stores:
        st.wait()


_sc_embed = pl.kernel(
    _embed_body,
    out_type=(
        jax.ShapeDtypeStruct((BATCH, SEQ_LEN, D_MODEL), jnp.float32),
        jax.ShapeDtypeStruct((BATCH, SEQ_LEN, D_MODEL), jnp.float32),
    ),
    mesh=plsc.VectorSubcoreMesh(core_axis_name="c", subcore_axis_name="s"),
    scratch_types=[
        pltpu.VMEM((_CHUNK,), jnp.int32),
        pltpu.VMEM((_CHUNK,), jnp.int32),
        pltpu.VMEM((_CHUNK, D_MODEL), jnp.float32),
        pltpu.VMEM((_CHUNK, D_MODEL), jnp.float32),
        pltpu.VMEM((_CHUNK, D_MODEL), jnp.float32),
    ] + [pltpu.SemaphoreType.DMA] * (3 + 2 * _NSUB),
)


@jax.jit
def kernel(x, x_output, W):
    pe = jnp.asarray(_PE)
    return _sc_embed(W, x, x_output, pe)


# speedup vs baseline: 1.0639x; 1.0039x over previous
import jax
import jax.numpy as jnp
import numpy as np
from jax import lax
from jax.experimental import pallas as pl
from jax.experimental.pallas import tpu as pltpu
from jax.experimental.pallas import tpu_sc as plsc

VOCAB = 100000
D_MODEL = 128
SEQ_LEN = 2048
BATCH = 4

_NC = 2
_NS = 16
_NW = _NC * _NS
_B_FLAT = BATCH * SEQ_LEN
_CHUNK = _B_FLAT // _NW
_NSUB = 2
_SUB = _CHUNK // _NSUB
_DV = D_MODEL // 16


def _build_pe() -> np.ndarray:
    pos = np.arange(SEQ_LEN)[:, None].astype(np.float64)
    i = np.arange(D_MODEL)[None, :].astype(np.float64)
    angle = pos / np.power(10000.0, (2.0 * (i // 2)) / D_MODEL)
    pe = np.zeros((SEQ_LEN, D_MODEL), dtype=np.float32)
    pe[:, 0::2] = np.sin(angle[:, 0::2])
    pe[:, 1::2] = np.cos(angle[:, 1::2])
    return pe


_PE = _build_pe()


def _embed_body(w_hbm, x1_hbm, x2_hbm, pe_hbm, out1_hbm, out2_hbm,
                idx1_v, idx2_v, pe_v, rows1_v, rows2_v, *sems):
    wid = lax.axis_index("s") * _NC + lax.axis_index("c")
    base = wid * _CHUNK
    b = base // SEQ_LEN
    s0 = lax.rem(base, SEQ_LEN)

    cp_pe = pltpu.async_copy(pe_hbm.at[pl.ds(s0, _CHUNK)], pe_v, sems[0])
    cp_i1 = pltpu.async_copy(x1_hbm.at[b, pl.ds(s0, _CHUNK)], idx1_v, sems[1])
    cp_i2 = pltpu.async_copy(x2_hbm.at[b, pl.ds(s0, _CHUNK)], idx2_v, sems[2])
    cp_i1.wait()
    cp_i2.wait()

    gathers = []
    for j in range(_NSUB):
        sl = pl.ds(_SUB * j, _SUB)
        for idx_v, rows_v in ((idx1_v, rows1_v), (idx2_v, rows2_v)):
            sem = sems[3 + len(gathers)]
            gathers.append(
                (pltpu.async_copy(w_hbm.at[idx_v.at[sl]], rows_v.at[sl], sem),
                 sem))

    cp_pe.wait()
    stores = []
    for j in range(_NSUB):
        sl = pl.ds(_SUB * j, _SUB)
        (cp1, sem1), (cp2, sem2) = gathers[2 * j], gathers[2 * j + 1]
        cp1.wait()
        cp2.wait()

        @plsc.parallel_loop(_SUB * j, _SUB * (j + 1), step=4)
        def _(r):
            for rr in range(4):
                for d in range(_DV):
                    dsl = pl.ds(d * 16, 16)
                    pv = pe_v[r + rr, dsl]
                    plsc.addupdate(rows1_v.at[r + rr, dsl], pv)
                    plsc.addupdate(rows2_v.at[r + rr, dsl], pv)

        o0 = s0 + _SUB * j
        stores.append(pltpu.async_copy(
            rows1_v.at[sl], out1_hbm.at[b, pl.ds(o0, _SUB)], sem1))
        stores.append(pltpu.async_copy(
            rows2_v.at[sl], out2_hbm.at[b, pl.ds(o0, _SUB)], sem2))
    for st in stores:
        st.wait()


_sc_embed = pl.kernel(
    _embed_body,
    out_type=(
        jax.ShapeDtypeStruct((BATCH, SEQ_LEN, D_MODEL), jnp.float32),
        jax.ShapeDtypeStruct((BATCH, SEQ_LEN, D_MODEL), jnp.float32),
    ),
    mesh=plsc.VectorSubcoreMesh(core_axis_name="c", subcore_axis_name="s"),
    scratch_types=[
        pltpu.VMEM((_CHUNK,), jnp.int32),
        pltpu.VMEM((_CHUNK,), jnp.int32),
        pltpu.VMEM((_CHUNK, D_MODEL), jnp.float32),
        pltpu.VMEM((_CHUNK, D_MODEL), jnp.float32),
        pltpu.VMEM((_CHUNK, D_MODEL), jnp.float32),
    ] + [pltpu.SemaphoreType.DMA] * (3 + 2 * _NSUB),
)


@jax.jit
def kernel(x, x_output, W):
    pe = jnp.asarray(_PE)
    return _sc_embed(W, x, x_output, pe)
